# Initial kernel scaffold; baseline (speedup 1.0000x reference)
import jax
import jax.numpy as jnp
from jax.experimental import pallas as pl


def kernel(x, edge_index, batch_size, params):
    raise NotImplementedError("write your pallas kernel here")


# speedup vs baseline: 3.2180x; 3.2180x over previous
import functools

import jax
import jax.numpy as jnp
from jax import lax
from jax.experimental import pallas as pl
from jax.experimental.pallas import tpu as pltpu
from jax.experimental.pallas import tpu_sc as plsc

F32 = jnp.float32
I32 = jnp.int32

N = 10000
E = 320000
H = 256
NC, NS, NW = 2, 16, 32
NP = N + 32
NPA = 10240
NPB = 80
EP = 327680
ECH = EP // 128
NEG = float(jnp.finfo(jnp.float32).min)

_MESH = dict(core_axis_name="c", subcore_axis_name="s",
             num_cores=NC, num_subcores=NS)


def _mo(v, m=8):
    return pl.multiple_of(v, m)


def _rmw(acc, liot, cc, val, op):
    rw = cc // 128
    ws = _mo(lax.rem(cc, 128) // 16 * 16, 16)
    ln = lax.rem(cc, 16)
    cur = acc[rw, pl.ds(ws, 16)]
    curval = jnp.sum(jnp.where(liot == ln, cur, jnp.zeros((16,), cur.dtype)))
    acc[rw, pl.ds(ws, 16)] = jnp.where(liot == ln, op(curval, val), cur)


def _sc_segsum(scaled, esplit=False):
    Dh = 128
    RT = ECH // (NW if esplit else NS)
    SB = 16 if scaled else 40
    NSTG = RT // SB
    RPT = NPA // NS
    ZR = 8

    outs = jax.ShapeDtypeStruct((2 * N, Dh), F32)
    scratch = [
        pltpu.VMEM((ZR, Dh), F32),
        pltpu.VMEM((SB, 128), I32),
        pltpu.VMEM((SB, 128), I32),
        pltpu.VMEM((128, Dh), F32),
        pltpu.VMEM_SHARED((NPA, Dh), F32),
        pltpu.SemaphoreType.DMA,
    ]
    if scaled:
        scratch += [
            pltpu.VMEM((SB, 128), F32),
        ]

    def body(*refs):
        if scaled:
            (x2, gidx, sidx, sc2d, out,
             zbuf, gv, sv, rows, acc, sem, ev) = refs
        else:
            (x2, gidx, sidx, out, zbuf, gv, sv, rows, acc, sem) = refs
        c = lax.axis_index("c")
        s = lax.axis_index("s")
        tid = (s * NC + c) if esplit else s

        nf = Dh // 16

        def zf(i, _):
            zbuf[i // nf, pl.ds((i % nf) * 16, 16)] = jnp.zeros((16,), F32)
            return 0
        lax.fori_loop(0, ZR * nf, zf, 0)
        base = s * RPT

        def zc(q, _):
            pltpu.sync_copy(zbuf, acc.at[pl.ds(_mo(base + q * ZR), ZR)])
            return 0
        lax.fori_loop(0, RPT // ZR, zc, 0)
        plsc.subcore_barrier()

        coff = c * N

        def stage(g, _):
            sbase = _mo(tid * RT + g * SB)
            pltpu.sync_copy(gidx.at[pl.ds(sbase, SB)], gv)
            pltpu.sync_copy(sidx.at[pl.ds(sbase, SB)], sv)
            if scaled:
                pltpu.sync_copy(sc2d.at[pl.ds(sbase, SB)], ev)

            if not esplit:
                def offb(i, _):
                    r, t = i // 8, i % 8
                    gv[r, pl.ds(t * 16, 16)] = gv[r, pl.ds(t * 16, 16)] + coff
                    return 0
                lax.fori_loop(0, SB * 8, offb, 0)

            def eb(j, _):
                pltpu.async_copy(x2.at[gv.at[j]], rows, sem).wait()
                if scaled:
                    def sg(t, _):
                        sc16 = ev[j, pl.ds(t * 16, 16)]
                        for k in range(16):
                            scs = sc16[k]
                            for f in range(nf):
                                rows[t * 16 + k, pl.ds(f * 16, 16)] = (
                                    rows[t * 16 + k, pl.ds(f * 16, 16)] * scs)
                        return 0
                    lax.fori_loop(0, 8, sg, 0)
                pltpu.sync_copy(rows, acc.at[sv.at[j]], add=True)
                return 0
            lax.fori_loop(0, SB, eb, 0)
            return 0
        lax.fori_loop(0, NSTG, stage, 0)
        plsc.subcore_barrier()

        ob = s * RPT

        @pl.when(s < NS - 1)
        def _():
            pltpu.sync_copy(acc.at[pl.ds(_mo(ob), RPT)],
                            out.at[pl.ds(_mo(c * N + ob), RPT)])

        @pl.when(s == NS - 1)
        def _():
            pltpu.sync_copy(acc.at[pl.ds(_mo(ob), N - (NS - 1) * RPT)],
                            out.at[pl.ds(_mo(c * N + ob), N - (NS - 1) * RPT)])

    return pl.kernel(body, out_type=outs,
                     mesh=plsc.VectorSubcoreMesh(**_MESH),
                     compiler_params=pltpu.CompilerParams(
                         needs_layout_passes=False),
                     scratch_types=scratch)


def _sc_segmax():
    RNG = 312
    LAST = N - RNG * (NW - 1)
    DUMP = LAST
    STG = 16
    NSTG = ECH // STG
    RING = 2048
    DB = 64

    scratch = [
        pltpu.VMEM((STG, 128), I32),
        pltpu.VMEM((STG, 128), I32),
        pltpu.VMEM((RING,), I32),
        pltpu.VMEM((RING,), I32),
        pltpu.VMEM((DB, 256), F32),
        pltpu.VMEM((LAST + 1, 256), F32),
        pltpu.SemaphoreType.DMA,
    ]

    def body(x_hbm, row2d, col2d, out, rv, cv, ring_r, ring_l, rows, acc, sem):
        c = lax.axis_index("c")
        s = lax.axis_index("s")
        wid = s * NC + c
        lo = wid * RNG
        hi = jnp.where(wid == NW - 1, N, lo + RNG)

        def iacc(i, _):
            acc[i // 16, pl.ds((i % 16) * 16, 16)] = jnp.full((16,), NEG, F32)
            return 0
        lax.fori_loop(0, (LAST + 1) * 16, iacc, 0)

        def iring(i, _):
            ring_r[pl.ds(i * 16, 16)] = jnp.zeros((16,), I32)
            ring_l[pl.ds(i * 16, 16)] = jnp.full((16,), DUMP, I32)
            return 0
        lax.fori_loop(0, RING // 16, iring, 0)

        def drain(d):
            st = _mo(lax.rem(d, RING), 64)
            pltpu.async_copy(x_hbm.at[ring_r.at[pl.ds(st, DB)]], rows, sem).wait()

            def per_q(q, _):
                le16 = ring_l[pl.ds(_mo(st + q * 16, 16), 16)]
                for k in range(16):
                    lc = le16[k]
                    i = q * 16 + k

                    def per_f(f, _):
                        a = acc[lc, pl.ds(f * 16, 16)]
                        b = rows[i, pl.ds(f * 16, 16)]
                        acc[lc, pl.ds(f * 16, 16)] = jnp.maximum(a, b)
                        return 0
                    lax.fori_loop(0, 16, per_f, 0)
                return 0
            lax.fori_loop(0, DB // 16, per_q, 0)
            return d + DB

        def stage_loop(g, carry):
            pltpu.sync_copy(row2d.at[pl.ds(_mo(g * STG, 16), STG)], rv)
            pltpu.sync_copy(col2d.at[pl.ds(_mo(g * STG, 16), STG)], cv)

            def row_loop(r, carry):
                def grp(t, carry):
                    mcnt, done = carry
                    c16 = cv[r, pl.ds(t * 16, 16)]
                    r16 = rv[r, pl.ds(t * 16, 16)]
                    m = (c16 >= lo) & (c16 < hi)
                    lc16 = jnp.where(m, c16 - lo, DUMP)
                    cs = plsc.cumsum(m.astype(I32))
                    pos = lax.rem(mcnt + cs - 1, RING)
                    plsc.store_scatter(ring_r, [pos], r16, mask=m)
                    plsc.store_scatter(ring_l, [pos], lc16, mask=m)
                    mcnt = mcnt + jnp.sum(m.astype(I32))
                    done = lax.cond(mcnt - done >= DB, drain, lambda d: d, done)
                    return (mcnt, done)
                return lax.fori_loop(0, 8, grp, carry)
            return lax.fori_loop(0, STG, row_loop, carry)

        mcnt, done = lax.fori_loop(0, NSTG, stage_loop, (0, 0))
        posp = lax.rem(mcnt + lax.iota(I32, 16), RING)
        plsc.store_scatter(ring_r, [posp], jnp.zeros((16,), I32))
        plsc.store_scatter(ring_l, [posp], jnp.full((16,), DUMP, I32))
        lax.cond(mcnt - done > 0, drain, lambda d: d, done)

        @pl.when(wid < NW - 1)
        def _():
            pltpu.sync_copy(acc.at[pl.ds(0, RNG)], out.at[pl.ds(_mo(lo), RNG)])

        @pl.when(wid == NW - 1)
        def _():
            pltpu.sync_copy(acc.at[pl.ds(0, LAST)], out.at[pl.ds(_mo(lo), LAST)])

    return pl.kernel(body, out_type=jax.ShapeDtypeStruct((N, 256), F32),
                     mesh=plsc.VectorSubcoreMesh(**_MESH),
                     compiler_params=pltpu.CompilerParams(
                         needs_layout_passes=False),
                     scratch_types=scratch)


def _sc_escore():
    RT = ECH // NW

    scratch = [
        pltpu.VMEM((NP,), F32),
        pltpu.VMEM((NP,), F32),
        pltpu.VMEM((NPB, 128), F32),
        pltpu.VMEM((16, 128), I32),
        pltpu.VMEM((16, 128), I32),
        pltpu.VMEM((16, 128), F32),
    ]

    def body(qv_h, xw_h, gidx, sidx, s_out, smax_out, qv, xw, acc, gv, sv, sb):
        c = lax.axis_index("c")
        s = lax.axis_index("s")
        wid = s * NC + c
        pltpu.sync_copy(qv_h, qv)
        pltpu.sync_copy(xw_h, xw)

        def ia(i, _):
            acc[i // 8, pl.ds((i % 8) * 16, 16)] = jnp.full((16,), NEG, F32)
            return 0
        lax.fori_loop(0, NPB * 8, ia, 0)

        liot = lax.iota(I32, 16)

        def stg(g, _):
            sbase = _mo(wid * RT + g * 16, 16)
            pltpu.sync_copy(gidx.at[pl.ds(sbase, 16)], gv)
            pltpu.sync_copy(sidx.at[pl.ds(sbase, 16)], sv)

            def vec(i, _):
                r, t = i // 8, i % 8
                r16 = gv[r, pl.ds(t * 16, 16)]
                c16 = sv[r, pl.ds(t * 16, 16)]
                v = plsc.load_gather(qv, [c16]) + plsc.load_gather(xw, [r16])
                s16 = jnp.where(v >= 0, v, 0.2 * v)
                sb[r, pl.ds(t * 16, 16)] = s16
                for k in range(16):
                    _rmw(acc, liot, c16[k], s16[k], jnp.maximum)
                return 0
            lax.fori_loop(0, 16 * 8, vec, 0)
            pltpu.sync_copy(sb, s_out.at[pl.ds(sbase, 16)])
            return 0
        lax.fori_loop(0, RT // 16, stg, 0)
        pltpu.sync_copy(acc, smax_out.at[pl.ds(_mo(wid * NPB), NPB)])

    return pl.kernel(
        body,
        out_type=(jax.ShapeDtypeStruct((ECH, 128), F32),
                  jax.ShapeDtypeStruct((NW * NPB, 128), F32)),
        mesh=plsc.VectorSubcoreMesh(**_MESH),
                     compiler_params=pltpu.CompilerParams(
                         needs_layout_passes=False),
        scratch_types=scratch)


def _sc_esoft():
    RT = ECH // NW

    scratch = [
        pltpu.VMEM((NP,), F32),
        pltpu.VMEM((NPB, 128), F32),
        pltpu.VMEM((NPB, 128), F32),
        pltpu.VMEM((16, 128), I32),
        pltpu.VMEM((16, 128), F32),
        pltpu.VMEM((16, 128), F32),
    ]

    def body(s_h, smax_h, sidx, e_out, z_out, d_out, smax, zac, dac, sv, ssb, eb):
        c = lax.axis_index("c")
        s = lax.axis_index("s")
        wid = s * NC + c
        pltpu.sync_copy(smax_h, smax)

        def ia(i, _):
            zac[i // 8, pl.ds((i % 8) * 16, 16)] = jnp.zeros((16,), F32)
            dac[i // 8, pl.ds((i % 8) * 16, 16)] = jnp.zeros((16,), F32)
            return 0
        lax.fori_loop(0, NPB * 8, ia, 0)

        liot = lax.iota(I32, 16)
        add = lambda a, b: a + b

        def stg(g, _):
            sbase = _mo(wid * RT + g * 16, 16)
            pltpu.sync_copy(sidx.at[pl.ds(sbase, 16)], sv)
            pltpu.sync_copy(s_h.at[pl.ds(sbase, 16)], ssb)

            def vec(i, _):
                r, t = i // 8, i % 8
                c16 = sv[r, pl.ds(t * 16, 16)]
                s16 = ssb[r, pl.ds(t * 16, 16)]
                e16 = jnp.exp(s16 - plsc.load_gather(smax, [c16]))
                eb[r, pl.ds(t * 16, 16)] = e16
                for k in range(16):
                    _rmw(zac, liot, c16[k], e16[k], add)
                    _rmw(dac, liot, c16[k], 1.0, add)
                return 0
            lax.fori_loop(0, 16 * 8, vec, 0)
            pltpu.sync_copy(eb, e_out.at[pl.ds(sbase, 16)])
            return 0
        lax.fori_loop(0, RT // 16, stg, 0)
        pltpu.sync_copy(zac, z_out.at[pl.ds(_mo(wid * NPB), NPB)])
        pltpu.sync_copy(dac, d_out.at[pl.ds(_mo(wid * NPB), NPB)])

    return pl.kernel(
        body,
        out_type=(jax.ShapeDtypeStruct((ECH, 128), F32),
                  jax.ShapeDtypeStruct((NW * NPB, 128), F32),
                  jax.ShapeDtypeStruct((NW * NPB, 128), F32)),
        mesh=plsc.VectorSubcoreMesh(**_MESH),
                     compiler_params=pltpu.CompilerParams(
                         needs_layout_passes=False),
        scratch_types=scratch)


def _sc_score():
    RT = ECH // NW

    scratch = [
        pltpu.VMEM((NP,), F32),
        pltpu.VMEM((16, 128), I32),
        pltpu.VMEM((16, 128), F32),
        pltpu.VMEM((16, 128), F32),
    ]

    def body(e_h, rz_h, sidx, sc_out, rz, sv, ev, ob):
        c = lax.axis_index("c")
        s = lax.axis_index("s")
        wid = s * NC + c
        pltpu.sync_copy(rz_h, rz)

        def stg(g, _):
            sbase = _mo(wid * RT + g * 16, 16)
            pltpu.sync_copy(sidx.at[pl.ds(sbase, 16)], sv)
            pltpu.sync_copy(e_h.at[pl.ds(sbase, 16)], ev)

            def vec(i, _):
                r, t = i // 8, i % 8
                c16 = sv[r, pl.ds(t * 16, 16)]
                e16 = ev[r, pl.ds(t * 16, 16)]
                ob[r, pl.ds(t * 16, 16)] = e16 * plsc.load_gather(rz, [c16])
                return 0
            lax.fori_loop(0, 16 * 8, vec, 0)
            pltpu.sync_copy(ob, sc_out.at[pl.ds(sbase, 16)])
            return 0
        lax.fori_loop(0, RT // 16, stg, 0)

    return pl.kernel(body, out_type=jax.ShapeDtypeStruct((ECH, 128), F32),
                     mesh=plsc.VectorSubcoreMesh(**_MESH),
                     compiler_params=pltpu.CompilerParams(
                         needs_layout_passes=False),
                     scratch_types=scratch)


def _sc_segsum_scalar():
    RT = ECH // NW

    scratch = [
        pltpu.VMEM((NP,), F32),
        pltpu.VMEM((NPB, 128), F32),
        pltpu.VMEM((16, 128), I32),
        pltpu.VMEM((16, 128), I32),
    ]

    def body(av_h, gidx, sidx, p_out, av, acc, gv, sv):
        c = lax.axis_index("c")
        s = lax.axis_index("s")
        wid = s * NC + c
        pltpu.sync_copy(av_h, av)

        def ia(i, _):
            acc[i // 8, pl.ds((i % 8) * 16, 16)] = jnp.zeros((16,), F32)
            return 0
        lax.fori_loop(0, NPB * 8, ia, 0)

        liot = lax.iota(I32, 16)
        add = lambda a, b: a + b

        def stg(g, _):
            sbase = _mo(wid * RT + g * 16, 16)
            pltpu.sync_copy(gidx.at[pl.ds(sbase, 16)], gv)
            pltpu.sync_copy(sidx.at[pl.ds(sbase, 16)], sv)

            def vec(i, _):
                r, t = i // 8, i % 8
                r16 = gv[r, pl.ds(t * 16, 16)]
                c16 = sv[r, pl.ds(t * 16, 16)]
                v16 = plsc.load_gather(av, [r16])
                for k in range(16):
                    _rmw(acc, liot, c16[k], v16[k], add)
                return 0
            lax.fori_loop(0, 16 * 8, vec, 0)
            return 0
        lax.fori_loop(0, RT // 16, stg, 0)
        pltpu.sync_copy(acc, p_out.at[pl.ds(_mo(wid * NPB), NPB)])

    return pl.kernel(body, out_type=jax.ShapeDtypeStruct((NW * NPB, 128), F32),
                     mesh=plsc.VectorSubcoreMesh(**_MESH),
                     compiler_params=pltpu.CompilerParams(
                         needs_layout_passes=False),
                     scratch_types=scratch)


def _sc_sbuild():
    RT = ECH // NS
    HN = N // NC
    FL = HN * 128
    FLP = 645120
    ZW = 13440
    SPT = FLP // NS
    NPT = 313

    SB = 16
    scratch = [
        pltpu.VMEM((ZW,), F32),
        pltpu.VMEM((SB, 128), I32),
        pltpu.VMEM((SB, 128), I32),
        pltpu.VMEM((SB, 128), F32),
        pltpu.VMEM((NP,), I32),
        pltpu.VMEM((NP,), F32),
        pltpu.VMEM((128,), I32),
        pltpu.VMEM((128,), F32),
        pltpu.VMEM((16,), I32),
        pltpu.VMEM((16,), F32),
        pltpu.VMEM_SHARED((FLP,), F32),
    ]

    def body(score_h, inv_h, ss_h, gidx, sidx, out,
             zbuf, gv, sv, scv, inv, ssl, ib, vb, sib, svb, acc):
        c = lax.axis_index("c")
        s = lax.axis_index("s")

        def zf(i, _):
            zbuf[pl.ds(i * 16, 16)] = jnp.zeros((16,), F32)
            return 0
        lax.fori_loop(0, ZW // 16, zf, 0)

        def zc(k, _):
            pltpu.sync_copy(zbuf, acc.at[pl.ds(_mo(s * SPT + k * ZW, 128), ZW)])
            return 0
        lax.fori_loop(0, 3, zc, 0)
        plsc.subcore_barrier()

        pltpu.sync_copy(inv_h, inv)
        pltpu.sync_copy(ss_h, ssl)

        rlo = c * HN

        def stage(g, _):
            sbase = _mo(s * RT + g * SB, 16)
            pltpu.sync_copy(gidx.at[pl.ds(sbase, SB)], gv)
            pltpu.sync_copy(sidx.at[pl.ds(sbase, SB)], sv)
            pltpu.sync_copy(score_h.at[pl.ds(sbase, SB)], scv)

            def eb(j, _):
                def grp(t, _):
                    r16 = gv[j, pl.ds(t * 16, 16)]
                    c16 = sv[j, pl.ds(t * 16, 16)]
                    v16 = scv[j, pl.ds(t * 16, 16)]
                    tc = plsc.load_gather(inv, [c16])
                    ok = (tc < 100) & (r16 >= rlo) & (r16 < rlo + HN)
                    ib[pl.ds(t * 16, 16)] = jnp.where(
                        ok, (r16 - rlo) * 128 + tc, FL)
                    vb[pl.ds(t * 16, 16)] = jnp.where(ok, v16, 0.0)
                    return 0
                lax.fori_loop(0, 8, grp, 0)
                pltpu.sync_copy(vb, acc.at[ib], add=True)
                return 0
            lax.fori_loop(0, SB, eb, 0)
            return 0
        lax.fori_loop(0, RT // SB, stage, 0)

        nbase = rlo + s * NPT
        ngroups = 20

        def sg(g, _):
            i16 = nbase + g * 16 + lax.iota(I32, 16)
            ok = (i16 < nbase + NPT) & (i16 < rlo + HN)
            ci = jnp.where(ok, i16, N)
            tc = plsc.load_gather(inv, [ci])
            ok = ok & (tc < 100)
            sib[...] = jnp.where(ok, (i16 - rlo) * 128 + tc, FL)
            svb[...] = jnp.where(ok, plsc.load_gather(ssl, [ci]), 0.0)
            pltpu.sync_copy(svb, acc.at[sib], add=True)
            return 0
        lax.fori_loop(0, ngroups, sg, 0)
        plsc.subcore_barrier()

        pltpu.sync_copy(acc.at[pl.ds(_mo(s * SPT, 128), SPT)],
                        out.at[pl.ds(_mo(c * FLP + s * SPT, 128), SPT)])

    return pl.kernel(body, out_type=jax.ShapeDtypeStruct((2 * FLP,), F32),
                     mesh=plsc.VectorSubcoreMesh(**_MESH),
                     compiler_params=pltpu.CompilerParams(
                         needs_layout_passes=False),
                     scratch_types=scratch)


_BN = 2000


def _dot(a, b):
    return jnp.dot(a, b, preferred_element_type=F32)


def _tc_gin(xc):
    Dh = 128

    def body(xin, aL, aR, w1, b1, w2, b2, out, gout, oh, goh):
        if xc == 128:
            h = xin[...] + (aL[...] + aR[...])
        else:
            h = xin[...] + jnp.concatenate([aL[...], aR[...]], axis=1)
        h = jax.nn.gelu(_dot(h, w1[...]) + b1[...])
        o = _dot(h, w2[...]) + b2[...]
        g = jax.nn.gelu(o)
        out[...] = o
        gout[...] = g
        oh[0] = o[:, :128]
        oh[1] = o[:, 128:]
        goh[0] = g[:, :128]
        goh[1] = g[:, 128:]

    grid = N // _BN
    return pl.pallas_call(
        body,
        grid=(grid,),
        in_specs=[
            pl.BlockSpec((_BN, xc), lambda i: (i, 0)),
            pl.BlockSpec((_BN, Dh), lambda i: (i, 0)),
            pl.BlockSpec((_BN, Dh), lambda i: (i, 0)),
            pl.BlockSpec((xc, H), lambda i: (0, 0)),
            pl.BlockSpec((1, H), lambda i: (0, 0)),
            pl.BlockSpec((H, H), lambda i: (0, 0)),
            pl.BlockSpec((1, H), lambda i: (0, 0)),
        ],
        out_specs=[
            pl.BlockSpec((_BN, H), lambda i: (i, 0)),
            pl.BlockSpec((_BN, H), lambda i: (i, 0)),
            pl.BlockSpec((2, _BN, 128), lambda i: (0, i, 0)),
            pl.BlockSpec((2, _BN, 128), lambda i: (0, i, 0)),
        ],
        out_shape=[
            jax.ShapeDtypeStruct((N, H), F32),
            jax.ShapeDtypeStruct((N, H), F32),
            jax.ShapeDtypeStruct((2, N, 128), F32),
            jax.ShapeDtypeStruct((2, N, 128), F32),
        ],
    )


def _tc_node1():
    def body(mx, z1, lw, lb, wq, wx, ab, qv_o, xw_o, ss_o):
        xm = jnp.maximum(mx[...], z1[...])
        xq = _dot(xm, lw[...]) + lb[...]
        qv = _dot(xq, wq[...]) + ab[0, 0]
        xw = _dot(z1[...], wx[...])
        v = qv + xw
        qv_o[...] = qv
        xw_o[...] = xw
        ss_o[...] = jnp.where(v >= 0, v, 0.2 * v)

    grid = N // _BN
    return pl.pallas_call(
        body,
        grid=(grid,),
        in_specs=[
            pl.BlockSpec((_BN, H), lambda i: (i, 0)),
            pl.BlockSpec((_BN, H), lambda i: (i, 0)),
            pl.BlockSpec((H, H), lambda i: (0, 0)),
            pl.BlockSpec((1, H), lambda i: (0, 0)),
            pl.BlockSpec((H, 1), lambda i: (0, 0)),
            pl.BlockSpec((H, 1), lambda i: (0, 0)),
            pl.BlockSpec((1, 1), lambda i: (0, 0)),
        ],
        out_specs=[pl.BlockSpec((_BN, 1), lambda i: (i, 0))] * 3,
        out_shape=[jax.ShapeDtypeStruct((N, 1), F32)] * 3,
    )


def _tc_sprep():
    def body(smax_p, ss, smax_o, es_o):
        sm = jnp.max(smax_p[...][:, :N], axis=0)[:, None]
        smf = jnp.maximum(sm, ss[...])
        smax_o[...] = smf
        es_o[...] = jnp.exp(ss[...] - smf)

    return pl.pallas_call(
        body,
        out_shape=[jax.ShapeDtypeStruct((N, 1), F32)] * 2,
    )


def _tc_sfin():
    def body(z_p, d_p, es, rz_o, ssc_o, deg_o):
        z = jnp.sum(z_p[...][:, :N], axis=0)[:, None] + es[...]
        rz = 1.0 / (z + 1e-16)
        rz_o[...] = rz
        ssc_o[...] = es[...] * rz
        deg_o[...] = jnp.sum(d_p[...][:, :N], axis=0)[:, None]

    return pl.pallas_call(
        body,
        out_shape=[jax.ShapeDtypeStruct((N, 1), F32)] * 3,
    )


def _tc_leconv():
    def body(xnL, xnR, z1, ssc, w1, b1, w2, w3, b3, xn_o, av_o, bv_o, cv_o):
        xn = (jnp.concatenate([xnL[...], xnR[...]], axis=1)
              + z1[...] * ssc[...])
        xn_o[...] = xn
        av_o[...] = _dot(xn, w1[...]) + b1[0, 0]
        bv_o[...] = _dot(xn, w2[...])
        cv_o[...] = _dot(xn, w3[...]) + b3[0, 0]

    grid = N // _BN
    return pl.pallas_call(
        body,
        grid=(grid,),
        in_specs=[
            pl.BlockSpec((_BN, 128), lambda i: (i, 0)),
            pl.BlockSpec((_BN, 128), lambda i: (i, 0)),
            pl.BlockSpec((_BN, H), lambda i: (i, 0)),
            pl.BlockSpec((_BN, 1), lambda i: (i, 0)),
            pl.BlockSpec((H, 1), lambda i: (0, 0)),
            pl.BlockSpec((1, 1), lambda i: (0, 0)),
            pl.BlockSpec((H, 1), lambda i: (0, 0)),
            pl.BlockSpec((H, 1), lambda i: (0, 0)),
            pl.BlockSpec((1, 1), lambda i: (0, 0)),
        ],
        out_specs=[
            pl.BlockSpec((_BN, H), lambda i: (i, 0)),
            pl.BlockSpec((_BN, 1), lambda i: (i, 0)),
            pl.BlockSpec((_BN, 1), lambda i: (i, 0)),
            pl.BlockSpec((_BN, 1), lambda i: (i, 0)),
        ],
        out_shape=[
            jax.ShapeDtypeStruct((N, H), F32),
            jax.ShapeDtypeStruct((N, 1), F32),
            jax.ShapeDtypeStruct((N, 1), F32),
            jax.ShapeDtypeStruct((N, 1), F32),
        ],
    )


def _tc_fitness():
    def body(agg_p, av, bv, cv, deg, fit_o):
        agg = jnp.sum(agg_p[...][:, :N], axis=0)[:, None]
        logit = agg + av[...] - (deg[...] + 1.0) * bv[...] + cv[...]
        fit_o[...] = jax.nn.sigmoid(logit)

    return pl.pallas_call(
        body,
        out_shape=jax.ShapeDtypeStruct((N, 1), F32),
    )


def _tc_topk(k):
    def body(fit, inv_o, f_s):
        f_s[...] = fit[...]
        inv_o[...] = jnp.full((N,), k, I32)
        iota = lax.broadcasted_iota(I32, (N,), 0)

        def it(p, _):
            fv = f_s[...]
            m = jnp.max(fv)
            sel = jnp.min(jnp.where(fv == m, iota, N))
            msk = iota == sel
            inv_o[...] = jnp.where(msk, p, inv_o[...])
            f_s[...] = jnp.where(msk, NEG, fv)
            return 0
        lax.fori_loop(0, k, it, 0)

    return pl.pallas_call(
        body,
        out_shape=jax.ShapeDtypeStruct((N,), I32),
        scratch_shapes=[pltpu.VMEM((N,), F32)],
    )


def _tc_permgather():
    def body(inv, xn, fit, out):
        i = pl.program_id(0)
        iot = lax.broadcasted_iota(I32, (128, _BN), 0)
        P = jnp.where((inv[...][:, 0][None, :] == iot) & (iot < 100), 1.0, 0.0)
        contrib = _dot(P, xn[...] * fit[...])

        @pl.when(i == 0)
        def _():
            out[...] = jnp.zeros_like(out)
        out[...] += contrib

    grid = N // _BN
    return pl.pallas_call(
        body,
        grid=(grid,),
        in_specs=[
            pl.BlockSpec((_BN, 1), lambda i: (i, 0)),
            pl.BlockSpec((_BN, H), lambda i: (i, 0)),
            pl.BlockSpec((_BN, 1), lambda i: (i, 0)),
        ],
        out_specs=pl.BlockSpec((128, H), lambda i: (0, 0)),
        out_shape=jax.ShapeDtypeStruct((128, H), F32),
    )


def _tc_ac():
    def body(S, t0, t1, out, accs):
        i = pl.program_id(0)
        Tb = t0[...] + t1[...] + S[...]

        @pl.when(i == 0)
        def _():
            accs[...] = jnp.zeros_like(accs)
        accs[...] += lax.dot_general(S[...], Tb, (((0,), (0,)), ((), ())),
                                     preferred_element_type=F32)

        @pl.when(i == N // _BN - 1)
        def _():
            r = lax.broadcasted_iota(I32, (128, 128), 0)
            cc = lax.broadcasted_iota(I32, (128, 128), 1)
            out[...] = jnp.where(r == cc, 1.0, accs[...])

    grid = N // _BN
    return pl.pallas_call(
        body,
        grid=(grid,),
        in_specs=[
            pl.BlockSpec((_BN, 128), lambda i: (i, 0)),
            pl.BlockSpec((_BN, 128), lambda i: (i, 0)),
            pl.BlockSpec((_BN, 128), lambda i: (i, 0)),
        ],
        out_specs=pl.BlockSpec((128, 128), lambda i: (0, 0)),
        out_shape=jax.ShapeDtypeStruct((128, 128), F32),
        scratch_shapes=[pltpu.VMEM((128, 128), F32)],
    )


def _tc_coarse():
    K = 100

    def body(zp1, Ac, w1a, b1a, w2a, b2a, w1b, b1b, w2b, b2b,
             lw, lb, wq, wx, ab, lew1, leb1, lew2, lew3, leb3,
             pw1b, pw1c, pb1, cvec_o, xq_s, mf_s, x_s):
        z = zp1[...]
        A = Ac[...]
        mf = jnp.where(A != 0.0, 1.0, 0.0)

        def gin(xx, w1, b1, w2, b2):
            h = xx + _dot(mf.T, xx)
            h = jax.nn.gelu(_dot(h, w1[...]) + b1[...])
            return _dot(h, w2[...]) + b2[...]

        hp = gin(z, w1a, b1a, w2a, b2a)
        x = gin(jax.nn.gelu(hp), w1b, b1b, w2b, b2b)

        xq_s[...] = jnp.full((K, H), NEG, F32)
        mf_s[...] = mf
        x_s[...] = x

        def mj(j, _):
            mrow = jnp.reshape(mf_s[pl.ds(j, 1), :], (K, 1))
            xrow = x_s[pl.ds(j, 1), :]
            contrib = jnp.where(mrow > 0, xrow, NEG)
            xq_s[...] = jnp.maximum(xq_s[...], contrib)
            return 0
        lax.fori_loop(0, K, mj, 0)

        xq = _dot(xq_s[...], lw[...]) + lb[...]
        s2 = (_dot(x, wx[...])[:, 0][:, None]
              + _dot(xq, wq[...])[:, 0][None, :] + ab[0, 0])
        s2 = jnp.where(s2 >= 0, s2, 0.2 * s2)
        msk = A != 0.0
        s2 = jnp.where(msk, s2, NEG)
        s2 = s2 - jnp.max(s2, axis=0, keepdims=True)
        e2 = jnp.where(msk, jnp.exp(s2), 0.0)
        score = e2 / (jnp.sum(e2, axis=0, keepdims=True) + 1e-16)
        xn2 = _dot(score.T, x)
        a2 = _dot(xn2, lew1[...]) + leb1[0, 0]
        b2v = _dot(xn2, lew2[...])
        agg2 = _dot(mf.T, a2) - jnp.sum(mf, axis=0)[:, None] * b2v
        fit2 = jax.nn.sigmoid((agg2 + _dot(xn2, lew3[...]) + leb3[0, 0])[:, 0])
        m = jnp.max(fit2)
        iot = lax.iota(I32, K)
        sel = jnp.min(jnp.where(fit2 == m, iot, K))
        oh = jnp.where(iot == sel, 1.0, 0.0)[None, :]
        zp2 = _dot(oh, xn2) * m

        gmax = jnp.max(z, axis=0, keepdims=True) + zp2
        gmean = jnp.mean(z, axis=0, keepdims=True) + zp2
        cvec_o[...] = _dot(gmax, pw1b[...]) + _dot(gmean, pw1c[...]) + pb1[...]

    return pl.pallas_call(
        body,
        out_shape=jax.ShapeDtypeStruct((1, H), F32),
        scratch_shapes=[pltpu.VMEM((K, H), F32),
                        pltpu.VMEM((K, K), F32),
                        pltpu.VMEM((K, H), F32)],
    )


def _tc_pred():
    def body(z2, cvec, w1a, w2, b2, out):
        hh = jax.nn.gelu(_dot(z2[...], w1a[...]) + cvec[...])
        out[...] = _dot(hh, w2[...]) + b2[...]

    grid = N // _BN
    return pl.pallas_call(
        body,
        grid=(grid,),
        in_specs=[
            pl.BlockSpec((_BN, H), lambda i: (i, 0)),
            pl.BlockSpec((1, H), lambda i: (0, 0)),
            pl.BlockSpec((H, H), lambda i: (0, 0)),
            pl.BlockSpec((H, 128), lambda i: (0, 0)),
            pl.BlockSpec((1, 128), lambda i: (0, 0)),
        ],
        out_specs=pl.BlockSpec((_BN, 128), lambda i: (i, 0)),
        out_shape=jax.ShapeDtypeStruct((N, 128), F32),
    )


def _r2(v):
    return v.reshape(1, -1)


def kernel(x, edge_index, batch_size, params):
    p = params
    row = edge_index[0].astype(I32)
    col = edge_index[1].astype(I32)
    padz = jnp.zeros((EP - E,), I32)
    padn = jnp.full((EP - E,), N, I32)
    g2dA = jnp.concatenate([row, padz]).reshape(ECH, 128)
    s2dA = jnp.concatenate([col, padn]).reshape(ECH, 128)
    g2dB = jnp.concatenate([col, padz]).reshape(ECH, 128)
    s2dB = jnp.concatenate([row, padn]).reshape(ECH, 128)

    segsum_e = _sc_segsum(False, esplit=True)
    segsum128 = _sc_segsum(False)
    segsum128s = _sc_segsum(True)
    gin128 = _tc_gin(128)
    gin256 = _tc_gin(256)

    def pad16(v, val=0.0):
        return jnp.pad(v, (0, NP - N), constant_values=val)

    agg1 = segsum_e(x, g2dA, s2dA)
    _, gz, _, gzh = gin128(x, agg1[:N], agg1[N:],
                           p["gin1a"]["W1"], _r2(p["gin1a"]["b1"]),
                           p["gin1a"]["W2"], _r2(p["gin1a"]["b2"]))
    agg1b = segsum128(gzh.reshape(2 * N, 128), g2dA, s2dA)
    z1, gz1, z1h, gz1h = gin256(gz, agg1b[:N], agg1b[N:],
                                p["gin1b"]["W1"], _r2(p["gin1b"]["b1"]),
                                p["gin1b"]["W2"], _r2(p["gin1b"]["b2"]))

    agg2a = segsum128(gz1h.reshape(2 * N, 128), g2dA, s2dA)
    _, gh2, _, gh2h = gin256(gz1, agg2a[:N], agg2a[N:],
                             p["gin2a"]["W1"], _r2(p["gin2a"]["b1"]),
                             p["gin2a"]["W2"], _r2(p["gin2a"]["b2"]))
    agg2b = segsum128(gh2h.reshape(2 * N, 128), g2dA, s2dA)
    z2 = gin256(gh2, agg2b[:N], agg2b[N:],
                p["gin2b"]["W1"], _r2(p["gin2b"]["b1"]),
                p["gin2b"]["W2"], _r2(p["gin2b"]["b2"]))[0]

    pool = p["pool1"]
    wq = pool["att_W"][:H]
    wx = pool["att_W"][H:]
    mx = _sc_segmax()(z1, g2dA, s2dA)
    qv, xw, ss = _tc_node1()(mx, z1, pool["lin_W"], _r2(pool["lin_b"]),
                             wq, wx, pool["att_b"].reshape(1, 1))
    s_e, smax_p = _sc_escore()(pad16(qv[:, 0]), pad16(xw[:, 0]), g2dA, s2dA)
    smaxf, e_self = _tc_sprep()(smax_p.reshape(NW, NPB * 128), ss)
    e_e, z_p, d_p = _sc_esoft()(s_e, pad16(smaxf[:, 0]), s2dA)
    rz, ssc, deg = _tc_sfin()(z_p.reshape(NW, NPB * 128),
                              d_p.reshape(NW, NPB * 128), e_self)
    score = _sc_score()(e_e, pad16(rz[:, 0]), s2dA)
    xn_p = segsum128s(z1h.reshape(2 * N, 128), g2dA, s2dA, score)
    xn, av, bv, cv = _tc_leconv()(xn_p[:N], xn_p[N:], z1, ssc,
                                  pool["le_W1"], pool["le_b1"].reshape(1, 1),
                                  pool["le_W2"], pool["le_W3"],
                                  pool["le_b3"].reshape(1, 1))
    agg_p = _sc_segsum_scalar()(pad16(av[:, 0]), g2dA, s2dA)
    fit = _tc_fitness()(agg_p.reshape(NW, NPB * 128), av, bv, cv, deg)
    inv = _tc_topk(100)(fit[:, 0])
    zp1p = _tc_permgather()(inv[:, None], xn, fit)

    S2 = _sc_sbuild()(score,
                      jnp.pad(inv, (0, NP - N), constant_values=100),
                      pad16(ssc[:, 0]), g2dA, s2dA)
    Sf = S2.reshape(2, 645120)[:, :640000].reshape(N, 128)
    T = segsum_e(Sf, g2dB, s2dB)
    Acp = _tc_ac()(Sf, T[:N], T[N:])

    pool2 = p["pool2"]
    pr = p["pred"]
    cvec = _tc_coarse()(
        zp1p[:100], Acp[:100, :100],
        p["gin2a"]["W1"], _r2(p["gin2a"]["b1"]),
        p["gin2a"]["W2"], _r2(p["gin2a"]["b2"]),
        p["gin2b"]["W1"], _r2(p["gin2b"]["b1"]),
        p["gin2b"]["W2"], _r2(p["gin2b"]["b2"]),
        pool2["lin_W"], _r2(pool2["lin_b"]),
        pool2["att_W"][:H], pool2["att_W"][H:], pool2["att_b"].reshape(1, 1),
        pool2["le_W1"], pool2["le_b1"].reshape(1, 1), pool2["le_W2"],
        pool2["le_W3"], pool2["le_b3"].reshape(1, 1),
        pr["W1"][H:2 * H], pr["W1"][2 * H:], _r2(pr["b1"]))
    return _tc_pred()(z2, cvec, pr["W1"][:H], pr["W2"], _r2(pr["b2"]))
